# baseline (device time: 821880 ns/iter reference)
import functools

import jax
import jax.numpy as jnp
from jax import lax
from jax.experimental import pallas as pl
from jax.experimental.pallas import tpu as pltpu

N_DEV = 16
R_HOPS = 8
L_HOPS = 7


def kernel(x, w_mat):
    x = x.astype(jnp.bfloat16)
    w = w_mat.astype(jnp.bfloat16)
    m_per, k = x.shape
    _, n_per = w.shape

    def body(x_ref, w_ref, out_ref, comm_r, comm_l,
             send_r, recv_r, send_l, recv_l, credit_r, credit_l):
        my = lax.axis_index("i")
        right = lax.rem(my + 1, N_DEV)
        left = lax.rem(my + N_DEV - 1, N_DEV)

        barrier = pltpu.get_barrier_semaphore()
        for nbr in (left, right):
            pl.semaphore_signal(barrier, inc=1, device_id=(nbr,),
                                device_id_type=pl.DeviceIdType.MESH)
        pl.semaphore_wait(barrier, 2)

        def compute(chunk, origin):
            y = jnp.dot(chunk, w_ref[:, :], preferred_element_type=jnp.float32)
            out_ref[pl.ds(origin * m_per, m_per), :] = jnp.maximum(y, 0.0)

        for h in range(R_HOPS):
            if h >= 2:
                pl.semaphore_wait(credit_r, 1)
            src_r = x_ref if h == 0 else comm_r.at[(h - 1) % 2]
            r_rdma = pltpu.make_async_remote_copy(
                src_ref=src_r,
                dst_ref=comm_r.at[h % 2],
                send_sem=send_r.at[h % 2],
                recv_sem=recv_r.at[h % 2],
                device_id=(right,),
                device_id_type=pl.DeviceIdType.MESH,
            )
            r_rdma.start()

            l_rdma = None
            if h < L_HOPS:
                if h >= 2:
                    pl.semaphore_wait(credit_l, 1)
                src_l = x_ref if h == 0 else comm_l.at[(h - 1) % 2]
                l_rdma = pltpu.make_async_remote_copy(
                    src_ref=src_l,
                    dst_ref=comm_l.at[h % 2],
                    send_sem=send_l.at[h % 2],
                    recv_sem=recv_l.at[h % 2],
                    device_id=(left,),
                    device_id_type=pl.DeviceIdType.MESH,
                )
                l_rdma.start()

            if h == 0:
                compute(x_ref[:, :], my)

            r_rdma.wait_send()
            if 1 <= h <= R_HOPS - 2:
                pl.semaphore_signal(credit_r, inc=1, device_id=(left,),
                                    device_id_type=pl.DeviceIdType.MESH)
            r_rdma.wait_recv()
            compute(comm_r[h % 2, :, :], lax.rem(my + N_DEV - h - 1, N_DEV))

            if h < L_HOPS:
                l_rdma.wait_send()
                if 1 <= h <= L_HOPS - 2:
                    pl.semaphore_signal(credit_l, inc=1, device_id=(right,),
                                        device_id_type=pl.DeviceIdType.MESH)
                l_rdma.wait_recv()
                compute(comm_l[h % 2, :, :], lax.rem(my + h + 1, N_DEV))

    return pl.pallas_call(
        body,
        out_shape=jax.ShapeDtypeStruct((N_DEV * m_per, n_per), jnp.float32),
        in_specs=[
            pl.BlockSpec(memory_space=pltpu.VMEM),
            pl.BlockSpec(memory_space=pltpu.VMEM),
        ],
        out_specs=pl.BlockSpec(memory_space=pltpu.VMEM),
        scratch_shapes=[
            pltpu.VMEM((2, m_per, k), jnp.bfloat16),
            pltpu.VMEM((2, m_per, k), jnp.bfloat16),
            pltpu.SemaphoreType.DMA((2,)),
            pltpu.SemaphoreType.DMA((2,)),
            pltpu.SemaphoreType.DMA((2,)),
            pltpu.SemaphoreType.DMA((2,)),
            pltpu.SemaphoreType.REGULAR,
            pltpu.SemaphoreType.REGULAR,
        ],
        compiler_params=pltpu.CompilerParams(collective_id=0),
    )(x, w)


# device time: 732755 ns/iter; 1.1216x vs baseline; 1.1216x over previous
import jax
import jax.numpy as jnp
from jax import lax
from jax.experimental import pallas as pl
from jax.experimental.pallas import tpu as pltpu

N_DEV = 16
N_HOPS = 8

RING = [0, 1, 5, 9, 13, 14, 10, 6, 2, 3, 7, 11, 15, 12, 8, 4]
POS = [0] * N_DEV
for _p, _d in enumerate(RING):
    POS[_d] = _p


def kernel(x, w_mat):
    x = x.astype(jnp.bfloat16)
    w = w_mat.astype(jnp.bfloat16)
    m_per, k = x.shape
    _, n_per = w.shape
    half = m_per // 2

    my = lax.axis_index("i")
    ring = jnp.asarray(RING, dtype=jnp.int32)
    pos = jnp.asarray(POS, dtype=jnp.int32)[my]
    hops = jnp.arange(N_HOPS, dtype=jnp.int32)
    meta = jnp.concatenate([
        ring[(pos + 1) % N_DEV][None],
        ring[(pos - 1) % N_DEV][None],
        ring[(pos - 1 - hops) % N_DEV],
        ring[(pos + 1 + hops) % N_DEV],
    ]).astype(jnp.int32)

    def body(meta_ref, x_ref, w_ref, out_ref, comm_r, comm_l,
             send_r, recv_r, send_l, recv_l, credit_r, credit_l):
        my_dev = lax.axis_index("i")
        right = meta_ref[0]
        left = meta_ref[1]

        barrier = pltpu.get_barrier_semaphore()
        for nbr in (left, right):
            pl.semaphore_signal(barrier, inc=1, device_id=(nbr,),
                                device_id_type=pl.DeviceIdType.MESH)
        pl.semaphore_wait(barrier, 2)

        def compute(chunk, origin, row_off=0, rows=m_per):
            y = jnp.dot(chunk, w_ref[:, :], preferred_element_type=jnp.float32)
            out_ref[pl.ds(origin * m_per + row_off, rows), :] = (
                jnp.maximum(y, 0.0))

        def make(src, dst_slot_rows, sems_s, sems_r, slot, nbr):
            return pltpu.make_async_remote_copy(
                src_ref=src, dst_ref=dst_slot_rows,
                send_sem=sems_s.at[slot], recv_sem=sems_r.at[slot],
                device_id=(nbr,), device_id_type=pl.DeviceIdType.MESH,
            )

        for h in range(N_HOPS):
            slot, prev = h % 2, (h - 1) % 2
            if h >= 2:
                pl.semaphore_wait(credit_r, 1)
            if h == 0:
                r_rdma = make(x_ref, comm_r.at[0], send_r, recv_r, 0, right)
            elif h < N_HOPS - 1:
                r_rdma = make(comm_r.at[prev], comm_r.at[slot],
                              send_r, recv_r, slot, right)
            else:
                r_rdma = make(comm_r.at[prev, pl.ds(0, half), :],
                              comm_r.at[slot, pl.ds(0, half), :],
                              send_r, recv_r, slot, right)
            r_rdma.start()

            if h >= 2:
                pl.semaphore_wait(credit_l, 1)
            if h == 0:
                l_rdma = make(x_ref, comm_l.at[0], send_l, recv_l, 0, left)
            elif h < N_HOPS - 1:
                l_rdma = make(comm_l.at[prev], comm_l.at[slot],
                              send_l, recv_l, slot, left)
            else:
                l_rdma = make(comm_l.at[prev, pl.ds(half, half), :],
                              comm_l.at[slot, pl.ds(half, half), :],
                              send_l, recv_l, slot, left)
            l_rdma.start()

            if h == 0:
                compute(x_ref[:, :], my_dev)
            else:
                compute(comm_r[prev, :, :], meta_ref[2 + h - 1])
                compute(comm_l[prev, :, :], meta_ref[10 + h - 1])

            r_rdma.wait_send()
            if 1 <= h <= N_HOPS - 2:
                pl.semaphore_signal(credit_r, inc=1, device_id=(left,),
                                    device_id_type=pl.DeviceIdType.MESH)
            l_rdma.wait_send()
            if 1 <= h <= N_HOPS - 2:
                pl.semaphore_signal(credit_l, inc=1, device_id=(right,),
                                    device_id_type=pl.DeviceIdType.MESH)
            r_rdma.wait_recv()
            l_rdma.wait_recv()

        last = (N_HOPS - 1) % 2
        compute(comm_r[last, pl.ds(0, half), :], meta_ref[2 + N_HOPS - 1],
                row_off=0, rows=half)
        compute(comm_l[last, pl.ds(half, half), :], meta_ref[10 + N_HOPS - 1],
                row_off=half, rows=half)

    return pl.pallas_call(
        body,
        out_shape=jax.ShapeDtypeStruct((N_DEV * m_per, n_per), jnp.float32),
        in_specs=[
            pl.BlockSpec(memory_space=pltpu.SMEM),
            pl.BlockSpec(memory_space=pltpu.VMEM),
            pl.BlockSpec(memory_space=pltpu.VMEM),
        ],
        out_specs=pl.BlockSpec(memory_space=pltpu.VMEM),
        scratch_shapes=[
            pltpu.VMEM((2, m_per, k), jnp.bfloat16),
            pltpu.VMEM((2, m_per, k), jnp.bfloat16),
            pltpu.SemaphoreType.DMA((2,)),
            pltpu.SemaphoreType.DMA((2,)),
            pltpu.SemaphoreType.DMA((2,)),
            pltpu.SemaphoreType.DMA((2,)),
            pltpu.SemaphoreType.REGULAR,
            pltpu.SemaphoreType.REGULAR,
        ],
        compiler_params=pltpu.CompilerParams(
            collective_id=0, vmem_limit_bytes=100 * 1024 * 1024),
    )(meta, x, w)


# device time: 719982 ns/iter; 1.1415x vs baseline; 1.0177x over previous
import jax
import jax.numpy as jnp
from jax import lax
from jax.experimental import pallas as pl
from jax.experimental.pallas import tpu as pltpu

N_DEV = 16
N_HOPS = 8

RING = [0, 1, 5, 9, 13, 14, 10, 6, 2, 3, 7, 11, 15, 12, 8, 4]
POS = [0] * N_DEV
for _p, _d in enumerate(RING):
    POS[_d] = _p

META = []
for _d in range(N_DEV):
    _p = POS[_d]
    META.append(
        [RING[(_p + 1) % N_DEV], RING[(_p - 1) % N_DEV]]
        + [RING[(_p - 1 - _h) % N_DEV] for _h in range(N_HOPS)]
        + [RING[(_p + 1 + _h) % N_DEV] for _h in range(N_HOPS)]
    )


def kernel(x, w_mat):
    w = w_mat.astype(jnp.bfloat16)
    m_per, k = x.shape
    _, n_per = w.shape
    half = m_per // 2
    meta = jnp.asarray(META, dtype=jnp.int32)

    def body(meta_ref, x_ref, w_ref, out_ref, comm_r, comm_l,
             send_r, recv_r, send_l, recv_l, credit_r, credit_l):
        my_dev = lax.axis_index("i")
        right = meta_ref[my_dev, 0]
        left = meta_ref[my_dev, 1]

        barrier = pltpu.get_barrier_semaphore()
        for nbr in (left, right):
            pl.semaphore_signal(barrier, inc=1, device_id=(nbr,),
                                device_id_type=pl.DeviceIdType.MESH)
        pl.semaphore_wait(barrier, 2)

        x_bf = x_ref[:, :].astype(jnp.bfloat16)
        comm_r[0, :, :] = x_bf
        comm_l[0, :, :] = x_bf

        def compute(chunk, origin, row_off=0, rows=m_per):
            y = jnp.dot(chunk, w_ref[:, :], preferred_element_type=jnp.float32)
            out_ref[pl.ds(origin * m_per + row_off, rows), :] = (
                jnp.maximum(y, 0.0))

        def make(src, dst, sems_s, sems_r, slot, nbr):
            return pltpu.make_async_remote_copy(
                src_ref=src, dst_ref=dst,
                send_sem=sems_s.at[slot], recv_sem=sems_r.at[slot],
                device_id=(nbr,), device_id_type=pl.DeviceIdType.MESH,
            )

        for h in range(N_HOPS):
            src, dst = h % 2, (h + 1) % 2
            if h >= 1:
                pl.semaphore_wait(credit_r, 1)
            if h < N_HOPS - 1:
                r_rdma = make(comm_r.at[src], comm_r.at[dst],
                              send_r, recv_r, dst, right)
            else:
                r_rdma = make(comm_r.at[src, pl.ds(0, half), :],
                              comm_r.at[dst, pl.ds(0, half), :],
                              send_r, recv_r, dst, right)
            r_rdma.start()

            if h >= 1:
                pl.semaphore_wait(credit_l, 1)
            if h < N_HOPS - 1:
                l_rdma = make(comm_l.at[src], comm_l.at[dst],
                              send_l, recv_l, dst, left)
            else:
                l_rdma = make(comm_l.at[src, pl.ds(half, half), :],
                              comm_l.at[dst, pl.ds(half, half), :],
                              send_l, recv_l, dst, left)
            l_rdma.start()

            if h == 0:
                compute(comm_r[0, :, :], my_dev)
            else:
                compute(comm_r[src, :, :], meta_ref[my_dev, 2 + h - 1])
                compute(comm_l[src, :, :], meta_ref[my_dev, 10 + h - 1])

            r_rdma.wait_send()
            if h <= N_HOPS - 2:
                pl.semaphore_signal(credit_r, inc=1, device_id=(left,),
                                    device_id_type=pl.DeviceIdType.MESH)
            l_rdma.wait_send()
            if h <= N_HOPS - 2:
                pl.semaphore_signal(credit_l, inc=1, device_id=(right,),
                                    device_id_type=pl.DeviceIdType.MESH)
            r_rdma.wait_recv()
            l_rdma.wait_recv()

        last = N_HOPS % 2
        compute(comm_r[last, pl.ds(0, half), :], meta_ref[my_dev, 2 + N_HOPS - 1],
                row_off=0, rows=half)
        compute(comm_l[last, pl.ds(half, half), :],
                meta_ref[my_dev, 10 + N_HOPS - 1], row_off=half, rows=half)

    return pl.pallas_call(
        body,
        out_shape=jax.ShapeDtypeStruct((N_DEV * m_per, n_per), jnp.float32),
        in_specs=[
            pl.BlockSpec(memory_space=pltpu.SMEM),
            pl.BlockSpec(memory_space=pltpu.VMEM),
            pl.BlockSpec(memory_space=pltpu.VMEM),
        ],
        out_specs=pl.BlockSpec(memory_space=pltpu.VMEM),
        scratch_shapes=[
            pltpu.VMEM((2, m_per, k), jnp.bfloat16),
            pltpu.VMEM((2, m_per, k), jnp.bfloat16),
            pltpu.SemaphoreType.DMA((2,)),
            pltpu.SemaphoreType.DMA((2,)),
            pltpu.SemaphoreType.DMA((2,)),
            pltpu.SemaphoreType.DMA((2,)),
            pltpu.SemaphoreType.REGULAR,
            pltpu.SemaphoreType.REGULAR,
        ],
        compiler_params=pltpu.CompilerParams(
            collective_id=0, vmem_limit_bytes=100 * 1024 * 1024),
    )(meta, x, w)


# device time: 716965 ns/iter; 1.1463x vs baseline; 1.0042x over previous
import jax
import jax.numpy as jnp
from jax import lax
from jax.experimental import pallas as pl
from jax.experimental.pallas import tpu as pltpu

N_DEV = 16
N_HOPS = 8

RING = [0, 1, 5, 9, 13, 14, 10, 6, 2, 3, 7, 11, 15, 12, 8, 4]
POS = [0] * N_DEV
for _p, _d in enumerate(RING):
    POS[_d] = _p

META = []
for _d in range(N_DEV):
    _p = POS[_d]
    META.append(
        [RING[(_p + 1) % N_DEV], RING[(_p - 1) % N_DEV]]
        + [RING[(_p - 1 - _h) % N_DEV] for _h in range(N_HOPS)]
        + [RING[(_p + 1 + _h) % N_DEV] for _h in range(N_HOPS)]
    )


def kernel(x, w_mat):
    w = w_mat.astype(jnp.bfloat16)
    m_per, k = x.shape
    _, n_per = w.shape
    half = m_per // 2
    meta = jnp.asarray(META, dtype=jnp.int32)

    def body(meta_ref, x_ref, w_ref, out_ref, comm_r, comm_l,
             send_r, recv_r, send_l, recv_l, credit_r, credit_l):
        my_dev = lax.axis_index("i")
        right = meta_ref[my_dev, 0]
        left = meta_ref[my_dev, 1]

        barrier = pltpu.get_barrier_semaphore()
        for nbr in (left, right):
            pl.semaphore_signal(barrier, inc=1, device_id=(nbr,),
                                device_id_type=pl.DeviceIdType.MESH)
        pl.semaphore_wait(barrier, 2)

        comm_r[0, :, :] = x_ref[:, :].astype(jnp.bfloat16)

        def compute(chunk, origin, row_off=0, rows=m_per):
            y = jnp.dot(chunk, w_ref[:, :], preferred_element_type=jnp.float32)
            out_ref[pl.ds(origin * m_per + row_off, rows), :] = (
                jnp.maximum(y, 0.0).astype(jnp.bfloat16))

        def make(src, dst, sems_s, sems_r, slot, nbr):
            return pltpu.make_async_remote_copy(
                src_ref=src, dst_ref=dst,
                send_sem=sems_s.at[slot], recv_sem=sems_r.at[slot],
                device_id=(nbr,), device_id_type=pl.DeviceIdType.MESH,
            )

        for h in range(N_HOPS):
            src, dst = h % 2, (h + 1) % 2
            if h >= 1:
                pl.semaphore_wait(credit_r, 1)
            if h < N_HOPS - 1:
                r_rdma = make(comm_r.at[src], comm_r.at[dst],
                              send_r, recv_r, dst, right)
            else:
                r_rdma = make(comm_r.at[src, pl.ds(0, half), :],
                              comm_r.at[dst, pl.ds(0, half), :],
                              send_r, recv_r, dst, right)
            r_rdma.start()

            if h >= 1:
                pl.semaphore_wait(credit_l, 1)
            if h == 0:
                l_rdma = make(comm_r.at[0], comm_l.at[dst],
                              send_l, recv_l, dst, left)
            elif h < N_HOPS - 1:
                l_rdma = make(comm_l.at[src], comm_l.at[dst],
                              send_l, recv_l, dst, left)
            else:
                l_rdma = make(comm_l.at[src, pl.ds(half, half), :],
                              comm_l.at[dst, pl.ds(half, half), :],
                              send_l, recv_l, dst, left)
            l_rdma.start()

            if h == 0:
                compute(comm_r[0, :, :], my_dev)
            else:
                compute(comm_r[src, :, :], meta_ref[my_dev, 2 + h - 1])
                compute(comm_l[src, :, :], meta_ref[my_dev, 10 + h - 1])

            r_rdma.wait_send()
            if h == 0:
                l_rdma.wait_send()
                pl.semaphore_signal(credit_r, inc=1, device_id=(left,),
                                    device_id_type=pl.DeviceIdType.MESH)
                pl.semaphore_signal(credit_l, inc=1, device_id=(right,),
                                    device_id_type=pl.DeviceIdType.MESH)
            else:
                if h <= N_HOPS - 2:
                    pl.semaphore_signal(credit_r, inc=1, device_id=(left,),
                                        device_id_type=pl.DeviceIdType.MESH)
                l_rdma.wait_send()
                if h <= N_HOPS - 2:
                    pl.semaphore_signal(credit_l, inc=1, device_id=(right,),
                                        device_id_type=pl.DeviceIdType.MESH)
            r_rdma.wait_recv()
            l_rdma.wait_recv()

        last = N_HOPS % 2
        compute(comm_r[last, pl.ds(0, half), :], meta_ref[my_dev, 2 + N_HOPS - 1],
                row_off=0, rows=half)
        compute(comm_l[last, pl.ds(half, half), :],
                meta_ref[my_dev, 10 + N_HOPS - 1], row_off=half, rows=half)

    return pl.pallas_call(
        body,
        out_shape=jax.ShapeDtypeStruct((N_DEV * m_per, n_per), jnp.bfloat16),
        in_specs=[
            pl.BlockSpec(memory_space=pltpu.SMEM),
            pl.BlockSpec(memory_space=pltpu.VMEM),
            pl.BlockSpec(memory_space=pltpu.VMEM),
        ],
        out_specs=pl.BlockSpec(memory_space=pltpu.VMEM),
        scratch_shapes=[
            pltpu.VMEM((2, m_per, k), jnp.bfloat16),
            pltpu.VMEM((2, m_per, k), jnp.bfloat16),
            pltpu.SemaphoreType.DMA((2,)),
            pltpu.SemaphoreType.DMA((2,)),
            pltpu.SemaphoreType.DMA((2,)),
            pltpu.SemaphoreType.DMA((2,)),
            pltpu.SemaphoreType.REGULAR,
            pltpu.SemaphoreType.REGULAR,
        ],
        compiler_params=pltpu.CompilerParams(
            collective_id=0, vmem_limit_bytes=100 * 1024 * 1024),
    )(meta, x, w)


# device time: 706849 ns/iter; 1.1627x vs baseline; 1.0143x over previous
import jax
import jax.numpy as jnp
from jax import lax
from jax.experimental import pallas as pl
from jax.experimental.pallas import tpu as pltpu

N_DEV = 16
N_HOPS = 8

RING = [0, 1, 5, 9, 13, 14, 10, 6, 2, 3, 7, 11, 15, 12, 8, 4]
POS = [0] * N_DEV
for _p, _d in enumerate(RING):
    POS[_d] = _p

META = []
for _d in range(N_DEV):
    _p = POS[_d]
    META.append(
        [RING[(_p + 1) % N_DEV], RING[(_p - 1) % N_DEV]]
        + [RING[(_p - 1 - _h) % N_DEV] for _h in range(N_HOPS)]
        + [RING[(_p + 1 + _h) % N_DEV] for _h in range(N_HOPS)]
    )


def kernel(x, w_mat):
    w = w_mat.astype(jnp.bfloat16)
    m_per, k = x.shape
    _, n_per = w.shape
    half = m_per // 2
    meta = jnp.asarray(META, dtype=jnp.int32)

    def body(meta_ref, x_ref, w_ref, out_ref, c_ra, c_rb, c_la, c_lb,
             send_sems, recv_sems, cred_ra, cred_rb, cred_la, cred_lb):
        my_dev = lax.axis_index("i")
        right = meta_ref[my_dev, 0]
        left = meta_ref[my_dev, 1]

        barrier = pltpu.get_barrier_semaphore()
        for nbr in (left, right):
            pl.semaphore_signal(barrier, inc=1, device_id=(nbr,),
                                device_id_type=pl.DeviceIdType.MESH)
        pl.semaphore_wait(barrier, 2)

        chains = {
            "rA": dict(buf=c_ra, idx=0, nbr=right, cred=cred_ra,
                       cred_to=left, base=2, off=0, last=7, src0=c_ra),
            "rB": dict(buf=c_rb, idx=1, nbr=right, cred=cred_rb,
                       cred_to=left, base=2, off=half, last=6, src0=c_rb),
            "lA": dict(buf=c_la, idx=2, nbr=left, cred=cred_la,
                       cred_to=right, base=10, off=0, last=6, src0=c_ra),
            "lB": dict(buf=c_lb, idx=3, nbr=left, cred=cred_lb,
                       cred_to=right, base=10, off=half, last=7, src0=c_rb),
        }
        pending = {}

        def compute(chunk, origin, off):
            y = jnp.dot(chunk, w_ref[:, :], preferred_element_type=jnp.float32)
            out_ref[pl.ds(origin * m_per + off, half), :] = (
                jnp.maximum(y, 0.0).astype(jnp.bfloat16))

        def issue(name, q):
            ch = chains[name]
            src = ch["src0"] if q == 0 else ch["buf"]
            rdma = pltpu.make_async_remote_copy(
                src_ref=src.at[q % 2],
                dst_ref=ch["buf"].at[(q + 1) % 2],
                send_sem=send_sems.at[ch["idx"], (q + 1) % 2],
                recv_sem=recv_sems.at[ch["idx"], (q + 1) % 2],
                device_id=(ch["nbr"],),
                device_id_type=pl.DeviceIdType.MESH,
            )
            rdma.start()
            pending[name] = rdma

        def drain(name, q):
            ch = chains[name]
            pending[name].wait_recv()
            compute(ch["buf"][q % 2, :, :], meta_ref[my_dev, ch["base"] + q - 1],
                    ch["off"])

        def credit(name, m):
            ch = chains[name]
            if m <= ch["last"] - 1:
                pl.semaphore_signal(ch["cred"], inc=1,
                                    device_id=(ch["cred_to"],),
                                    device_id_type=pl.DeviceIdType.MESH)

        c_ra[0, :, :] = x_ref[pl.ds(0, half), :].astype(jnp.bfloat16)
        issue("rA", 0)
        issue("lA", 0)
        compute(c_ra[0, :, :], my_dev, 0)
        c_rb[0, :, :] = x_ref[pl.ds(half, half), :].astype(jnp.bfloat16)
        issue("rB", 0)
        issue("lB", 0)
        compute(c_rb[0, :, :], my_dev, half)

        for j in range(2, 14):
            q = j // 2
            r, l = ("rA", "lA") if j % 2 == 0 else ("rB", "lB")
            drain(r, q)
            drain(l, q)
            pending[r].wait_send()
            pending[l].wait_send()
            credit(r, q - 1)
            credit(l, q - 1)
            pl.semaphore_wait(chains[r]["cred"], 1)
            issue(r, q)
            pl.semaphore_wait(chains[l]["cred"], 1)
            issue(l, q)

        drain("rA", 7)
        drain("lA", 7)
        pending["rA"].wait_send()
        pending["lA"].wait_send()
        credit("rA", 6)
        credit("lA", 6)
        pl.semaphore_wait(cred_ra, 1)
        issue("rA", 7)
        drain("rB", 7)
        drain("lB", 7)
        pending["rB"].wait_send()
        pending["lB"].wait_send()
        credit("rB", 6)
        credit("lB", 6)
        pl.semaphore_wait(cred_lb, 1)
        issue("lB", 7)

        pending["rA"].wait_recv()
        compute(c_ra[0, :, :], meta_ref[my_dev, 2 + 7], 0)
        pending["lB"].wait_recv()
        compute(c_lb[0, :, :], meta_ref[my_dev, 10 + 7], half)
        pending["rA"].wait_send()
        pending["lB"].wait_send()

    return pl.pallas_call(
        body,
        out_shape=jax.ShapeDtypeStruct((N_DEV * m_per, n_per), jnp.bfloat16),
        in_specs=[
            pl.BlockSpec(memory_space=pltpu.SMEM),
            pl.BlockSpec(memory_space=pltpu.VMEM),
            pl.BlockSpec(memory_space=pltpu.VMEM),
        ],
        out_specs=pl.BlockSpec(memory_space=pltpu.VMEM),
        scratch_shapes=[
            pltpu.VMEM((2, half, k), jnp.bfloat16),
            pltpu.VMEM((2, half, k), jnp.bfloat16),
            pltpu.VMEM((2, half, k), jnp.bfloat16),
            pltpu.VMEM((2, half, k), jnp.bfloat16),
            pltpu.SemaphoreType.DMA((4, 2)),
            pltpu.SemaphoreType.DMA((4, 2)),
            pltpu.SemaphoreType.REGULAR,
            pltpu.SemaphoreType.REGULAR,
            pltpu.SemaphoreType.REGULAR,
            pltpu.SemaphoreType.REGULAR,
        ],
        compiler_params=pltpu.CompilerParams(
            collective_id=0, vmem_limit_bytes=100 * 1024 * 1024),
    )(meta, x, w)


# device time: 703245 ns/iter; 1.1687x vs baseline; 1.0051x over previous
import jax
import jax.numpy as jnp
from jax import lax
from jax.experimental import pallas as pl
from jax.experimental.pallas import tpu as pltpu

N_DEV = 16
N_HOPS = 8

RING = [0, 1, 5, 9, 13, 14, 10, 6, 2, 3, 7, 11, 15, 12, 8, 4]
POS = [0] * N_DEV
for _p, _d in enumerate(RING):
    POS[_d] = _p

META = []
for _d in range(N_DEV):
    _p = POS[_d]
    META.append(
        [RING[(_p + 1) % N_DEV], RING[(_p - 1) % N_DEV]]
        + [RING[(_p - 1 - _h) % N_DEV] for _h in range(N_HOPS)]
        + [RING[(_p + 1 + _h) % N_DEV] for _h in range(N_HOPS)]
    )


def kernel(x, w_mat):
    m_per, k = x.shape
    _, n_per = w_mat.shape
    half = m_per // 2
    meta = jnp.asarray(META, dtype=jnp.int32)

    def body(meta_ref, x_ref, w_ref, out_ref, c_ra, c_rb, c_la, c_lb,
             stage, w_bf, send_sems, recv_sems,
             stage_sem, cred_ra, cred_rb, cred_la, cred_lb):
        my_dev = lax.axis_index("i")
        right = meta_ref[my_dev, 0]
        left = meta_ref[my_dev, 1]

        barrier = pltpu.get_barrier_semaphore()
        for nbr in (left, right):
            pl.semaphore_signal(barrier, inc=1, device_id=(nbr,),
                                device_id_type=pl.DeviceIdType.MESH)
        pl.semaphore_wait(barrier, 2)

        chains = {
            "rA": dict(buf=c_ra, idx=0, nbr=right, cred=cred_ra,
                       cred_to=left, base=2, off=0, last=7, src0=c_ra),
            "rB": dict(buf=c_rb, idx=1, nbr=right, cred=cred_rb,
                       cred_to=left, base=2, off=half, last=6, src0=c_rb),
            "lA": dict(buf=c_la, idx=2, nbr=left, cred=cred_la,
                       cred_to=right, base=10, off=0, last=6, src0=c_ra),
            "lB": dict(buf=c_lb, idx=3, nbr=left, cred=cred_lb,
                       cred_to=right, base=10, off=half, last=7, src0=c_rb),
        }
        pending = {}

        def compute(chunk, origin, off):
            y = jnp.dot(chunk, w_bf[:, :], preferred_element_type=jnp.float32)
            out_ref[pl.ds(origin * m_per + off, half), :] = (
                jnp.maximum(y, 0.0).astype(jnp.bfloat16))

        def issue(name, q):
            ch = chains[name]
            src = ch["src0"] if q == 0 else ch["buf"]
            rdma = pltpu.make_async_remote_copy(
                src_ref=src.at[q % 2],
                dst_ref=ch["buf"].at[(q + 1) % 2],
                send_sem=send_sems.at[ch["idx"], (q + 1) % 2],
                recv_sem=recv_sems.at[ch["idx"], (q + 1) % 2],
                device_id=(ch["nbr"],),
                device_id_type=pl.DeviceIdType.MESH,
            )
            rdma.start()
            pending[name] = rdma

        def drain(name, q):
            ch = chains[name]
            pending[name].wait_recv()
            compute(ch["buf"][q % 2, :, :], meta_ref[my_dev, ch["base"] + q - 1],
                    ch["off"])

        def credit(name, m):
            ch = chains[name]
            if m <= ch["last"] - 1:
                pl.semaphore_signal(ch["cred"], inc=1,
                                    device_id=(ch["cred_to"],),
                                    device_id_type=pl.DeviceIdType.MESH)

        cp_a = pltpu.make_async_copy(x_ref.at[pl.ds(0, half), :], stage,
                                     stage_sem)
        cp_a.start()
        cp_a.wait()
        c_ra[0, :, :] = stage[:, :].astype(jnp.bfloat16)
        issue("rA", 0)
        issue("lA", 0)
        cp_b = pltpu.make_async_copy(x_ref.at[pl.ds(half, half), :], stage,
                                     stage_sem)
        cp_b.start()
        w_bf[:, :] = w_ref[:, :].astype(jnp.bfloat16)
        cp_b.wait()
        c_rb[0, :, :] = stage[:, :].astype(jnp.bfloat16)
        issue("rB", 0)
        issue("lB", 0)
        compute(c_ra[0, :, :], my_dev, 0)
        compute(c_rb[0, :, :], my_dev, half)

        for j in range(2, 14):
            q = j // 2
            r, l = ("rA", "lA") if j % 2 == 0 else ("rB", "lB")
            drain(r, q)
            drain(l, q)
            pending[r].wait_send()
            pending[l].wait_send()
            credit(r, q - 1)
            credit(l, q - 1)
            pl.semaphore_wait(chains[r]["cred"], 1)
            issue(r, q)
            pl.semaphore_wait(chains[l]["cred"], 1)
            issue(l, q)

        drain("rA", 7)
        drain("lA", 7)
        pending["rA"].wait_send()
        pending["lA"].wait_send()
        credit("rA", 6)
        credit("lA", 6)
        pl.semaphore_wait(cred_ra, 1)
        issue("rA", 7)
        drain("rB", 7)
        drain("lB", 7)
        pending["rB"].wait_send()
        pending["lB"].wait_send()
        credit("rB", 6)
        credit("lB", 6)
        pl.semaphore_wait(cred_lb, 1)
        issue("lB", 7)

        pending["rA"].wait_recv()
        compute(c_ra[0, :, :], meta_ref[my_dev, 2 + 7], 0)
        pending["lB"].wait_recv()
        compute(c_lb[0, :, :], meta_ref[my_dev, 10 + 7], half)
        pending["rA"].wait_send()
        pending["lB"].wait_send()

    return pl.pallas_call(
        body,
        out_shape=jax.ShapeDtypeStruct((N_DEV * m_per, n_per), jnp.bfloat16),
        in_specs=[
            pl.BlockSpec(memory_space=pltpu.SMEM),
            pl.BlockSpec(memory_space=pl.ANY),
            pl.BlockSpec(memory_space=pltpu.VMEM),
        ],
        out_specs=pl.BlockSpec(memory_space=pltpu.VMEM),
        scratch_shapes=[
            pltpu.VMEM((2, half, k), jnp.bfloat16),
            pltpu.VMEM((2, half, k), jnp.bfloat16),
            pltpu.VMEM((2, half, k), jnp.bfloat16),
            pltpu.VMEM((2, half, k), jnp.bfloat16),
            pltpu.VMEM((half, k), jnp.float32),
            pltpu.VMEM((k, n_per), jnp.bfloat16),
            pltpu.SemaphoreType.DMA((4, 2)),
            pltpu.SemaphoreType.DMA((4, 2)),
            pltpu.SemaphoreType.DMA,
            pltpu.SemaphoreType.REGULAR,
            pltpu.SemaphoreType.REGULAR,
            pltpu.SemaphoreType.REGULAR,
            pltpu.SemaphoreType.REGULAR,
        ],
        compiler_params=pltpu.CompilerParams(
            collective_id=0, vmem_limit_bytes=100 * 1024 * 1024),
    )(meta, x, w_mat)
